# baseline (speedup 1.0000x reference)
import functools

import jax
import jax.numpy as jnp
from jax import lax
from jax.experimental import pallas as pl
from jax.experimental.pallas import tpu as pltpu
from jax.experimental.pallas import tpu_sc as plsc

_B, _S, _D, _E = 4, 8192, 768, 64
_NW = 32
_RPW = (_B * _S) // _NW
_RCHUNK = 64
_NITER = _RPW // _RCHUNK
_NV = _D // 16

_mesh = plsc.VectorSubcoreMesh(
    core_axis_name="c", subcore_axis_name="s", num_cores=2, num_subcores=16
)


@functools.partial(
    pl.kernel,
    out_type=jax.ShapeDtypeStruct((_NW, _D), jnp.float32),
    mesh=_mesh,
    scratch_types=[
        pltpu.VMEM((2, _RCHUNK * _D), jnp.float32),
        pltpu.VMEM((_D,), jnp.float32),
        pltpu.SemaphoreType.DMA,
        pltpu.SemaphoreType.DMA,
    ],
)
def _sc_mean(x_hbm, out_hbm, buf, accv, sem0, sem1):
    wid = lax.axis_index("s") * 2 + lax.axis_index("c")
    base = wid * (_RPW * _D)
    sems = (sem0, sem1)

    def dma(i, k):
        return pltpu.make_async_copy(
            x_hbm.at[pl.ds(base + i * (_RCHUNK * _D), _RCHUNK * _D)],
            buf.at[k],
            sems[k],
        )

    dma(0, 0).start()
    zero = jnp.zeros((16,), jnp.float32)
    for j in range(_NV):
        accv[pl.ds(j * 16, 16)] = zero
    for i in range(_NITER):
        k = i % 2
        if i + 1 < _NITER:
            dma(i + 1, 1 - k).start()
        dma(i, k).wait()

        def row_body(r, carry, k=k):
            off = r * _D
            for j in range(_NV):
                plsc.addupdate(
                    accv.at[pl.ds(j * 16, 16)], buf[k, pl.ds(off + j * 16, 16)]
                )
            return carry

        lax.fori_loop(0, _RCHUNK, row_body, 0)

    pltpu.sync_copy(accv, out_hbm.at[wid])


def _gate_tail(m, wh, bh, wo, bo):
    h = jnp.dot(m, wh, preferred_element_type=jnp.float32) + bh
    h = h * jax.nn.sigmoid(h)
    logits = jnp.dot(h, wo, preferred_element_type=jnp.float32) + bo
    iota = lax.broadcasted_iota(jnp.int32, logits.shape, 1)
    v1 = jnp.max(logits, axis=1, keepdims=True)
    i1 = jnp.min(jnp.where(logits == v1, iota, _E), axis=1, keepdims=True)
    masked = jnp.where(iota == i1, -jnp.inf, logits)
    v2 = jnp.max(masked, axis=1, keepdims=True)
    i2 = jnp.min(jnp.where(masked == v2, iota, _E), axis=1, keepdims=True)
    e2 = jnp.exp(v2 - v1)
    denom = 1.0 + e2
    w = jnp.concatenate([1.0 / denom, e2 / denom], axis=1)
    i = jnp.concatenate([i1, i2], axis=1)
    return w, i


def _tail_body(p_ref, wh_ref, bh_ref, wo_ref, bo_ref, w_ref, i_ref):
    p = p_ref[...]
    m = jnp.sum(p.reshape(_B, _NW // _B, _D), axis=1) * (1.0 / _S)
    w, i = _gate_tail(m, wh_ref[...], bh_ref[...], wo_ref[...], bo_ref[...])
    w_ref[...] = w
    i_ref[...] = i


def kernel(x, W_hidden, b_hidden, W_out, b_out):
    partials = _sc_mean(x.reshape(-1))
    bh = b_hidden.reshape(1, _D)
    bo = b_out.reshape(1, _E)
    w, i = pl.pallas_call(
        _tail_body,
        out_shape=[
            jax.ShapeDtypeStruct((_B, 2), jnp.float32),
            jax.ShapeDtypeStruct((_B, 2), jnp.int32),
        ],
    )(partials, W_hidden, bh, W_out, bo)
    return w, i


# speedup vs baseline: 1.5334x; 1.5334x over previous
import functools

import jax
import jax.numpy as jnp
from jax import lax
from jax.experimental import pallas as pl
from jax.experimental.pallas import tpu as pltpu
from jax.experimental.pallas import tpu_sc as plsc

_B, _S, _D, _E = 4, 8192, 768, 64
_NW = 32
_RPW = (_B * _S) // _NW
_RCHUNK = 64
_NITER = _RPW // _RCHUNK
_NV = _D // 16
_UN = 8

_mesh = plsc.VectorSubcoreMesh(
    core_axis_name="c", subcore_axis_name="s", num_cores=2, num_subcores=16
)


@functools.partial(
    pl.kernel,
    out_type=jax.ShapeDtypeStruct((_NW, _D), jnp.float32),
    mesh=_mesh,
    scratch_types=[
        pltpu.VMEM((2, _RCHUNK * _D), jnp.float32),
        pltpu.VMEM((_D,), jnp.float32),
        pltpu.SemaphoreType.DMA,
        pltpu.SemaphoreType.DMA,
    ],
)
def _sc_mean(x_hbm, out_hbm, buf, accv, sem0, sem1):
    wid = lax.axis_index("s") * 2 + lax.axis_index("c")
    base = wid * (_RPW * _D)
    sems = (sem0, sem1)

    def dma(i, k):
        return pltpu.make_async_copy(
            x_hbm.at[pl.ds(base + i * (_RCHUNK * _D), _RCHUNK * _D)],
            buf.at[k],
            sems[k],
        )

    zero = jnp.zeros((16,), jnp.float32)
    for j in range(_NV):
        accv[pl.ds(j * 16, 16)] = zero

    def process(k):
        def jbody(j, c, k=k):
            off0 = j * 16

            def body(r8, a, k=k):
                off = r8 * (_UN * _D) + off0
                for u in range(_UN):
                    a = a + buf[k, pl.ds(off + u * _D, 16)]
                return a

            a = lax.fori_loop(0, _RCHUNK // _UN, body, accv[pl.ds(off0, 16)])
            accv[pl.ds(off0, 16)] = a
            return c

        lax.fori_loop(0, _NV, jbody, 0)

    dma(0, 0).start()
    dma(1, 1).start()

    def outer(i2, carry):
        i = i2 * 2
        dma(i, 0).wait()
        process(0)

        @pl.when(i + 2 < _NITER)
        def _():
            dma(i + 2, 0).start()

        dma(i + 1, 1).wait()
        process(1)

        @pl.when(i + 3 < _NITER)
        def _():
            dma(i + 3, 1).start()

        return carry

    lax.fori_loop(0, _NITER // 2, outer, 0)
    pltpu.sync_copy(accv, out_hbm.at[wid])


def _gate_tail(m, wh, bh, wo, bo):
    h = jnp.dot(m, wh, preferred_element_type=jnp.float32) + bh
    h = h * jax.nn.sigmoid(h)
    logits = jnp.dot(h, wo, preferred_element_type=jnp.float32) + bo
    iota = lax.broadcasted_iota(jnp.int32, logits.shape, 1)
    v1 = jnp.max(logits, axis=1, keepdims=True)
    i1 = jnp.min(jnp.where(logits == v1, iota, _E), axis=1, keepdims=True)
    masked = jnp.where(iota == i1, -jnp.inf, logits)
    v2 = jnp.max(masked, axis=1, keepdims=True)
    i2 = jnp.min(jnp.where(masked == v2, iota, _E), axis=1, keepdims=True)
    e2 = jnp.exp(v2 - v1)
    denom = 1.0 + e2
    w = jnp.concatenate([1.0 / denom, e2 / denom], axis=1)
    i = jnp.concatenate([i1, i2], axis=1)
    return w, i


def _tail_body(p_ref, wh_ref, bh_ref, wo_ref, bo_ref, w_ref, i_ref):
    p = p_ref[...]
    m = jnp.sum(p.reshape(_B, _NW // _B, _D), axis=1) * (1.0 / _S)
    w, i = _gate_tail(m, wh_ref[...], bh_ref[...], wo_ref[...], bo_ref[...])
    w_ref[...] = w
    i_ref[...] = i


def kernel(x, W_hidden, b_hidden, W_out, b_out):
    partials = _sc_mean(x.reshape(-1))
    bh = b_hidden.reshape(1, _D)
    bo = b_out.reshape(1, _E)
    w, i = pl.pallas_call(
        _tail_body,
        out_shape=[
            jax.ShapeDtypeStruct((_B, 2), jnp.float32),
            jax.ShapeDtypeStruct((_B, 2), jnp.int32),
        ],
    )(partials, W_hidden, bh, W_out, bo)
    return w, i


# speedup vs baseline: 2.1497x; 1.4019x over previous
import functools

import jax
import jax.numpy as jnp
from jax import lax
from jax.experimental import pallas as pl
from jax.experimental.pallas import tpu as pltpu
from jax.experimental.pallas import tpu_sc as plsc

_B, _S, _D, _E = 4, 8192, 768, 64
_NW = 32
_SC_ROWS_PB = 1536
_TC_ROWS_PB = _S - _SC_ROWS_PB
_WPB = _NW // _B
_RPW = _SC_ROWS_PB // _WPB
_RCHUNK = 24
_NITER = _RPW // _RCHUNK
_NV = _D // 16
_NBUF = 4
_CHUNK_TC = 1664

_mesh = plsc.VectorSubcoreMesh(
    core_axis_name="c", subcore_axis_name="s", num_cores=2, num_subcores=16
)


@functools.partial(
    pl.kernel,
    out_type=jax.ShapeDtypeStruct((_NW, _D), jnp.float32),
    mesh=_mesh,
    scratch_types=[
        pltpu.VMEM((_NBUF, _RCHUNK * _D), jnp.float32),
        pltpu.VMEM((_D,), jnp.float32),
        pltpu.SemaphoreType.DMA,
        pltpu.SemaphoreType.DMA,
        pltpu.SemaphoreType.DMA,
        pltpu.SemaphoreType.DMA,
    ],
)
def _sc_mean(x_hbm, out_hbm, buf, accv, sem0, sem1, sem2, sem3):
    wid = lax.axis_index("s") * 2 + lax.axis_index("c")
    b = wid // _WPB
    k_in_b = wid % _WPB
    base = (b * _S + _TC_ROWS_PB + k_in_b * _RPW) * _D
    sems = (sem0, sem1, sem2, sem3)

    def dma(i, k):
        return pltpu.make_async_copy(
            x_hbm.at[pl.ds(base + i * (_RCHUNK * _D), _RCHUNK * _D)],
            buf.at[k],
            sems[k],
        )

    zero = jnp.zeros((16,), jnp.float32)
    for j in range(_NV):
        accv[pl.ds(j * 16, 16)] = zero

    zerov = jnp.zeros((16,), jnp.float32)

    def process(k):
        def jbody(j, c, k=k):
            off0 = j * 16
            accs = [accv[pl.ds(off0, 16)], zerov, zerov, zerov]
            for r in range(_RCHUNK):
                accs[r % 4] = accs[r % 4] + buf[k, pl.ds(off0 + r * _D, 16)]
            accv[pl.ds(off0, 16)] = (accs[0] + accs[1]) + (accs[2] + accs[3])
            return c

        lax.fori_loop(0, _NV, jbody, 0)

    for p in range(_NBUF - 1):
        dma(p, p).start()

    def outer(i4, carry):
        i = i4 * _NBUF
        for p in range(_NBUF):
            dma(i + p, p).wait()
            process(p)

            @pl.when(i + p + _NBUF - 1 < _NITER)
            def _(i=i, p=p):
                dma(i + p + _NBUF - 1, (p + _NBUF - 1) % _NBUF).start()

        return carry

    lax.fori_loop(0, _NITER // _NBUF, outer, 0)
    pltpu.sync_copy(accv, out_hbm.at[wid])


def _gate_tail(m, wh, bh, wo, bo):
    h = jnp.dot(m, wh, preferred_element_type=jnp.float32) + bh
    h = h * jax.nn.sigmoid(h)
    logits = jnp.dot(h, wo, preferred_element_type=jnp.float32) + bo
    iota = lax.broadcasted_iota(jnp.int32, logits.shape, 1)
    v1 = jnp.max(logits, axis=1, keepdims=True)
    i1 = jnp.min(jnp.where(logits == v1, iota, _E), axis=1, keepdims=True)
    masked = jnp.where(iota == i1, -jnp.inf, logits)
    v2 = jnp.max(masked, axis=1, keepdims=True)
    i2 = jnp.min(jnp.where(masked == v2, iota, _E), axis=1, keepdims=True)
    e2 = jnp.exp(v2 - v1)
    denom = 1.0 + e2
    w = jnp.concatenate([1.0 / denom, e2 / denom], axis=1)
    i = jnp.concatenate([i1, i2], axis=1)
    return w, i


def _tc_reduce_body(x_ref, o_ref):
    b = pl.program_id(0)
    c = pl.program_id(1)
    partial = jnp.sum(x_ref[0], axis=0, keepdims=True)

    @pl.when(c == 0)
    def _():
        o_ref[pl.ds(b, 1), :] = partial

    @pl.when(c > 0)
    def _():
        o_ref[pl.ds(b, 1), :] += partial


def _tail_body(a_ref, p_ref, wh_ref, bh_ref, wo_ref, bo_ref, w_ref, i_ref):
    p = p_ref[...]
    m = (a_ref[...] + jnp.sum(p.reshape(_B, _WPB, _D), axis=1)) * (1.0 / _S)
    w, i = _gate_tail(m, wh_ref[...], bh_ref[...], wo_ref[...], bo_ref[...])
    w_ref[...] = w
    i_ref[...] = i


def kernel(x, W_hidden, b_hidden, W_out, b_out):
    partials = _sc_mean(x.reshape(-1))
    acc_tc = pl.pallas_call(
        _tc_reduce_body,
        grid=(_B, _TC_ROWS_PB // _CHUNK_TC),
        in_specs=[pl.BlockSpec((1, _CHUNK_TC, _D), lambda b, c: (b, c, 0))],
        out_specs=pl.BlockSpec((_B, _D), lambda b, c: (0, 0)),
        out_shape=jax.ShapeDtypeStruct((_B, _D), jnp.float32),
    )(x)
    bh = b_hidden.reshape(1, _D)
    bo = b_out.reshape(1, _E)
    w, i = pl.pallas_call(
        _tail_body,
        out_shape=[
            jax.ShapeDtypeStruct((_B, 2), jnp.float32),
            jax.ShapeDtypeStruct((_B, 2), jnp.int32),
        ],
    )(acc_tc, partials, W_hidden, bh, W_out, bo)
    return w, i


# speedup vs baseline: 8.1986x; 3.8139x over previous
import functools

import jax
import jax.numpy as jnp
from jax import lax
from jax.experimental import pallas as pl
from jax.experimental.pallas import tpu as pltpu
from jax.experimental.pallas import tpu_sc as plsc

_B, _S, _D, _E = 4, 8192, 768, 64
_NW = 32
_SC_ROWS_PB = 1536
_TC_ROWS_PB = _S - _SC_ROWS_PB
_WPB = _NW // _B
_RPW = _SC_ROWS_PB // _WPB
_RCHUNK = 24
_NITER = _RPW // _RCHUNK
_NV = _D // 16
_NBUF = 4
_CHUNK_TC = 2048

_mesh = plsc.VectorSubcoreMesh(
    core_axis_name="c", subcore_axis_name="s", num_cores=2, num_subcores=16
)


@functools.partial(
    pl.kernel,
    out_type=jax.ShapeDtypeStruct((_NW, _D), jnp.float32),
    mesh=_mesh,
    scratch_types=[
        pltpu.VMEM((_NBUF, _RCHUNK * _D), jnp.float32),
        pltpu.VMEM((_D,), jnp.float32),
        pltpu.SemaphoreType.DMA,
        pltpu.SemaphoreType.DMA,
        pltpu.SemaphoreType.DMA,
        pltpu.SemaphoreType.DMA,
    ],
)
def _sc_mean(x_hbm, out_hbm, buf, accv, sem0, sem1, sem2, sem3):
    wid = lax.axis_index("s") * 2 + lax.axis_index("c")
    b = wid // _WPB
    k_in_b = wid % _WPB
    base = (b * _S + _TC_ROWS_PB + k_in_b * _RPW) * _D
    sems = (sem0, sem1, sem2, sem3)

    def dma(i, k):
        return pltpu.make_async_copy(
            x_hbm.at[pl.ds(base + i * (_RCHUNK * _D), _RCHUNK * _D)],
            buf.at[k],
            sems[k],
        )

    zero = jnp.zeros((16,), jnp.float32)
    for j in range(_NV):
        accv[pl.ds(j * 16, 16)] = zero

    zerov = jnp.zeros((16,), jnp.float32)

    def process(k):
        def jbody(j, c, k=k):
            off0 = j * 16
            accs = [accv[pl.ds(off0, 16)], zerov, zerov, zerov]
            for r in range(_RCHUNK):
                accs[r % 4] = accs[r % 4] + buf[k, pl.ds(off0 + r * _D, 16)]
            accv[pl.ds(off0, 16)] = (accs[0] + accs[1]) + (accs[2] + accs[3])
            return c

        lax.fori_loop(0, _NV, jbody, 0)

    for p in range(_NBUF - 1):
        dma(p, p).start()

    def outer(i4, carry):
        i = i4 * _NBUF
        for p in range(_NBUF):
            dma(i + p, p).wait()
            process(p)

            @pl.when(i + p + _NBUF - 1 < _NITER)
            def _(i=i, p=p):
                dma(i + p + _NBUF - 1, (p + _NBUF - 1) % _NBUF).start()

        return carry

    lax.fori_loop(0, _NITER // _NBUF, outer, 0)
    pltpu.sync_copy(accv, out_hbm.at[wid])


def _gate_tail(m, wh, bh, wo, bo):
    h = jnp.dot(m, wh, preferred_element_type=jnp.float32) + bh
    h = h * jax.nn.sigmoid(h)
    logits = jnp.dot(h, wo, preferred_element_type=jnp.float32) + bo
    iota = lax.broadcasted_iota(jnp.int32, logits.shape, 1)
    v1 = jnp.max(logits, axis=1, keepdims=True)
    i1 = jnp.min(jnp.where(logits == v1, iota, _E), axis=1, keepdims=True)
    masked = jnp.where(iota == i1, -jnp.inf, logits)
    v2 = jnp.max(masked, axis=1, keepdims=True)
    i2 = jnp.min(jnp.where(masked == v2, iota, _E), axis=1, keepdims=True)
    e2 = jnp.exp(v2 - v1)
    denom = 1.0 + e2
    w = jnp.concatenate([1.0 / denom, e2 / denom], axis=1)
    i = jnp.concatenate([i1, i2], axis=1)
    return w, i


def _tc_reduce_body(x_ref, o_ref):
    b = pl.program_id(0)
    c = pl.program_id(1)
    partial = jnp.sum(x_ref[0], axis=0, keepdims=True)

    @pl.when(c == 0)
    def _():
        o_ref[pl.ds(b, 1), :] = partial

    @pl.when(c > 0)
    def _():
        o_ref[pl.ds(b, 1), :] += partial


def _tail_body(a_ref, wh_ref, bh_ref, wo_ref, bo_ref, w_ref, i_ref):
    m = a_ref[...] * (1.0 / _S)
    w, i = _gate_tail(m, wh_ref[...], bh_ref[...], wo_ref[...], bo_ref[...])
    w_ref[...] = w
    i_ref[...] = i


def kernel(x, W_hidden, b_hidden, W_out, b_out):
    acc = pl.pallas_call(
        _tc_reduce_body,
        grid=(_B, _S // _CHUNK_TC),
        in_specs=[pl.BlockSpec((1, _CHUNK_TC, _D), lambda b, c: (b, c, 0))],
        out_specs=pl.BlockSpec((_B, _D), lambda b, c: (0, 0)),
        out_shape=jax.ShapeDtypeStruct((_B, _D), jnp.float32),
    )(x)
    bh = b_hidden.reshape(1, _D)
    bo = b_out.reshape(1, _E)
    w, i = pl.pallas_call(
        _tail_body,
        out_shape=[
            jax.ShapeDtypeStruct((_B, 2), jnp.float32),
            jax.ShapeDtypeStruct((_B, 2), jnp.int32),
        ],
    )(acc, W_hidden, bh, W_out, bo)
    return w, i
